# Initial kernel scaffold; baseline (speedup 1.0000x reference)
import jax
import jax.numpy as jnp
from jax.experimental import pallas as pl


def kernel(node_type, ls, table):
    raise NotImplementedError("write your pallas kernel here")


# speedup vs baseline: 1.7786x; 1.7786x over previous
import functools

import jax
import jax.numpy as jnp
from jax import lax
from jax.experimental import pallas as pl
from jax.experimental.pallas import tpu as pltpu
from jax.experimental.pallas import tpu_sc as plsc

N_NODES = 100000
DIM = 128
TYPE_NUM = 300
LANES = 16

W = 80
NBLK = N_NODES // W
NC, NS = 2, 16
NW = NC * NS
ITERS = -(-NBLK // NW)


def _embed(table, node_type, ls):
    mesh = plsc.VectorSubcoreMesh(core_axis_name="c", subcore_axis_name="s")

    @functools.partial(
        pl.kernel,
        out_type=jax.ShapeDtypeStruct((N_NODES, DIM), jnp.float32),
        mesh=mesh,
        scratch_types=[
            pltpu.VMEM((W,), jnp.int32),
            pltpu.VMEM((W,), jnp.int32),
            pltpu.VMEM((W,), jnp.int32),
            pltpu.VMEM((W, DIM), jnp.float32),
            pltpu.SemaphoreType.DMA,
        ],
    )
    def k(t_hbm, nt_hbm, ls_hbm, out_hbm, nt_v, ls_v, idx_v, rows_v, sem):
        wid = lax.axis_index("s") * NC + lax.axis_index("c")

        @pl.loop(0, ITERS)
        def _(i):
            blk = wid + i * NW

            @pl.when(blk < NBLK)
            def _():
                base = blk * W
                pltpu.sync_copy(nt_hbm.at[pl.ds(base, W)], nt_v)
                pltpu.sync_copy(ls_hbm.at[pl.ds(base, W)], ls_v)

                @pl.loop(0, W, step=LANES)
                def _(j):
                    sl = pl.ds(j, LANES)
                    idx_v[sl] = nt_v[sl] + ls_v[sl] * 100

                pltpu.async_copy(t_hbm.at[idx_v], rows_v, sem).wait()
                pltpu.sync_copy(rows_v, out_hbm.at[pl.ds(base, W)])

    return k(table, node_type, ls)


def kernel(node_type, ls, table):
    t = table.at[0].set(0.0)
    nt = node_type.astype(jnp.int32)
    l = ls.astype(jnp.int32)
    return _embed(t, nt, l)


# speedup vs baseline: 2.5531x; 1.4354x over previous
import functools

import jax
import jax.numpy as jnp
from jax import lax
from jax.experimental import pallas as pl
from jax.experimental.pallas import tpu as pltpu
from jax.experimental.pallas import tpu_sc as plsc

N_NODES = 100000
DIM = 128
TYPE_NUM = 300
LANES = 16

W = 80
NBLK = N_NODES // W
NC, NS = 2, 16
NW = NC * NS
ITERS = -(-NBLK // NW)


def _embed(table, node_type, ls):
    mesh = plsc.VectorSubcoreMesh(core_axis_name="c", subcore_axis_name="s")

    @functools.partial(
        pl.kernel,
        out_type=jax.ShapeDtypeStruct((N_NODES, DIM), jnp.float32),
        mesh=mesh,
        scratch_types=[
            pltpu.VMEM_SHARED((TYPE_NUM, DIM), jnp.float32),
            pltpu.VMEM((W,), jnp.int32),
            pltpu.VMEM((W,), jnp.int32),
            pltpu.VMEM((W,), jnp.int32),
            pltpu.VMEM((W, DIM), jnp.float32),
            pltpu.SemaphoreType.DMA,
        ],
    )
    def k(t_hbm, nt_hbm, ls_hbm, out_hbm, tab_v, nt_v, ls_v, idx_v, rows_v, sem):
        wid = lax.axis_index("s") * NC + lax.axis_index("c")

        @pl.when(lax.axis_index("s") == 0)
        def _():
            pltpu.sync_copy(t_hbm, tab_v)

        plsc.subcore_barrier()

        @pl.loop(0, ITERS)
        def _(i):
            blk = wid + i * NW

            @pl.when(blk < NBLK)
            def _():
                base = blk * W
                pltpu.sync_copy(nt_hbm.at[pl.ds(base, W)], nt_v)
                pltpu.sync_copy(ls_hbm.at[pl.ds(base, W)], ls_v)

                @pl.loop(0, W, step=LANES)
                def _(j):
                    sl = pl.ds(j, LANES)
                    idx_v[sl] = nt_v[sl] + ls_v[sl] * 100

                pltpu.async_copy(tab_v.at[idx_v], rows_v, sem).wait()
                pltpu.sync_copy(rows_v, out_hbm.at[pl.ds(base, W)])

    return k(table, node_type, ls)


def kernel(node_type, ls, table):
    t = table.at[0].set(0.0)
    nt = node_type.astype(jnp.int32)
    l = ls.astype(jnp.int32)
    return _embed(t, nt, l)


# speedup vs baseline: 5.3612x; 2.0998x over previous
import functools

import jax
import jax.numpy as jnp
from jax import lax
from jax.experimental import pallas as pl
from jax.experimental.pallas import tpu as pltpu
from jax.experimental.pallas import tpu_sc as plsc

N_NODES = 100000
DIM = 128
TYPE_NUM = 300
LANES = 16

NC, NS = 2, 16
NW = NC * NS
CHUNK = 3200
W = 128
NBLK = CHUNK // W
NBUF = 5


def _embed(table, node_type, ls):
    mesh = plsc.VectorSubcoreMesh(core_axis_name="c", subcore_axis_name="s")

    @functools.partial(
        pl.kernel,
        out_type=jax.ShapeDtypeStruct((N_NODES, DIM), jnp.float32),
        mesh=mesh,
        scratch_types=[
            pltpu.VMEM_SHARED((TYPE_NUM, DIM), jnp.float32),
            pltpu.VMEM((CHUNK,), jnp.int32),
            pltpu.VMEM((CHUNK,), jnp.int32),
            pltpu.VMEM((CHUNK,), jnp.int32),
            [pltpu.VMEM((W, DIM), jnp.float32) for _ in range(NBUF)],
            [pltpu.SemaphoreType.DMA for _ in range(NBUF)],
            pltpu.SemaphoreType.DMA,
        ],
    )
    def k(t_hbm, nt_hbm, ls_hbm, out_hbm, tab_v, nt_v, ls_v, idx_v, rows,
          wsem, gsem):
        wid = lax.axis_index("s") * NC + lax.axis_index("c")

        @pl.when(lax.axis_index("s") == 0)
        def _():
            pltpu.sync_copy(t_hbm, tab_v)

        plsc.subcore_barrier()

        base = jnp.minimum(wid * CHUNK, N_NODES - CHUNK)
        pltpu.sync_copy(nt_hbm.at[pl.ds(base, CHUNK)], nt_v)
        pltpu.sync_copy(ls_hbm.at[pl.ds(base, CHUNK)], ls_v)

        @pl.loop(0, CHUNK, step=LANES)
        def _(j):
            sl = pl.ds(j, LANES)
            idx_v[sl] = nt_v[sl] + ls_v[sl] * 100

        @pl.loop(0, NBLK // NBUF)
        def _(i0):
            for b in range(NBUF):
                i = i0 * NBUF + b

                @pl.when(i0 > 0)
                def _():
                    pltpu.make_async_copy(
                        rows[b], out_hbm.at[pl.ds(0, W)], wsem[b]
                    ).wait()

                pltpu.async_copy(
                    tab_v.at[idx_v.at[pl.ds(i * W, W)]], rows[b], gsem
                ).wait()
                pltpu.async_copy(
                    rows[b], out_hbm.at[pl.ds(base + i * W, W)], wsem[b]
                )

        for b in range(NBUF):
            pltpu.make_async_copy(
                rows[b], out_hbm.at[pl.ds(0, W)], wsem[b]
            ).wait()

    return k(table, node_type, ls)


def kernel(node_type, ls, table):
    t = table.at[0].set(0.0)
    nt = node_type.astype(jnp.int32)
    l = ls.astype(jnp.int32)
    return _embed(t, nt, l)


# speedup vs baseline: 5.4395x; 1.0146x over previous
import functools

import jax
import jax.numpy as jnp
from jax import lax
from jax.experimental import pallas as pl
from jax.experimental.pallas import tpu as pltpu
from jax.experimental.pallas import tpu_sc as plsc

N_NODES = 100000
DIM = 128
TYPE_NUM = 300
LANES = 16

NC, NS = 2, 16
NW = NC * NS
CHUNK = 3200
W = 128
NBLK = CHUNK // W
NBUF = 5


def _embed(table, node_type, ls):
    mesh = plsc.VectorSubcoreMesh(core_axis_name="c", subcore_axis_name="s")

    @functools.partial(
        pl.kernel,
        out_type=jax.ShapeDtypeStruct((N_NODES, DIM), jnp.float32),
        mesh=mesh,
        scratch_types=[
            pltpu.VMEM_SHARED((TYPE_NUM, DIM), jnp.float32),
            pltpu.VMEM((CHUNK,), jnp.int32),
            pltpu.VMEM((CHUNK,), jnp.int32),
            pltpu.VMEM((CHUNK,), jnp.int32),
            [pltpu.VMEM((W, DIM), jnp.float32) for _ in range(NBUF)],
            [pltpu.SemaphoreType.DMA for _ in range(NBUF)],
            [pltpu.SemaphoreType.DMA for _ in range(NBUF)],
        ],
    )
    def k(t_hbm, nt_hbm, ls_hbm, out_hbm, tab_v, nt_v, ls_v, idx_v, rows,
          wsem, gsem):
        wid = lax.axis_index("s") * NC + lax.axis_index("c")

        @pl.when(lax.axis_index("s") == 0)
        def _():
            pltpu.sync_copy(t_hbm, tab_v)

        plsc.subcore_barrier()

        base = jnp.minimum(wid * CHUNK, N_NODES - CHUNK)
        pltpu.sync_copy(nt_hbm.at[pl.ds(base, CHUNK)], nt_v)
        pltpu.sync_copy(ls_hbm.at[pl.ds(base, CHUNK)], ls_v)

        @pl.loop(0, CHUNK, step=LANES)
        def _(j):
            sl = pl.ds(j, LANES)
            idx_v[sl] = nt_v[sl] + ls_v[sl] * 100

        def fire_gather(i, b):
            pltpu.async_copy(
                tab_v.at[idx_v.at[pl.ds(i * W, W)]], rows[b], gsem[b]
            )

        def wait_gather(i, b):
            pltpu.make_async_copy(
                tab_v.at[idx_v.at[pl.ds(i * W, W)]], rows[b], gsem[b]
            ).wait()

        @pl.loop(0, NBLK // NBUF)
        def _(i0):
            for b in range(NBUF):
                i = i0 * NBUF + b
                bp = (b - 1) % NBUF

                @pl.when(i0 > 0)
                def _():
                    pltpu.make_async_copy(
                        rows[b], out_hbm.at[pl.ds(0, W)], wsem[b]
                    ).wait()

                fire_gather(i, b)

                @pl.when((i0 > 0) | (b > 0))
                def _():
                    wait_gather(i - 1, bp)
                    pltpu.async_copy(
                        rows[bp], out_hbm.at[pl.ds(base + (i - 1) * W, W)],
                        wsem[bp],
                    )

        last = NBLK - 1
        lb = last % NBUF
        wait_gather(last, lb)
        pltpu.async_copy(
            rows[lb], out_hbm.at[pl.ds(base + last * W, W)], wsem[lb]
        )
        for b in range(NBUF):
            pltpu.make_async_copy(
                rows[b], out_hbm.at[pl.ds(0, W)], wsem[b]
            ).wait()

    return k(table, node_type, ls)


def kernel(node_type, ls, table):
    t = table.at[0].set(0.0)
    nt = node_type.astype(jnp.int32)
    l = ls.astype(jnp.int32)
    return _embed(t, nt, l)
